# Initial kernel scaffold; baseline (speedup 1.0000x reference)
import jax
import jax.numpy as jnp
from jax.experimental import pallas as pl


def kernel(input, weight):
    raise NotImplementedError("write your pallas kernel here")


# speedup vs baseline: 3.3460x; 3.3460x over previous
import functools

import jax
import jax.numpy as jnp
from jax import lax
from jax.experimental import pallas as pl

_TN = 1024


def _vq_block(x_ref, w_ref, d_ref, e_ref, q_ref, l_ref, *, hw, wdt, k):
    g = pl.program_id(0)
    x = x_ref[...]
    w = w_ref[...]
    d = (x * x + w * w) - 2.0 * (x * w)
    d_ref[...] = d
    mind = jnp.min(d, axis=1, keepdims=True)
    kiota = lax.broadcasted_iota(jnp.int32, d.shape, 1)
    idx = jnp.min(jnp.where(d == mind, kiota, k), axis=1, keepdims=True)
    n = g * _TN + lax.broadcasted_iota(jnp.int32, x.shape, 0)
    h = (n % hw) // wdt
    lim = jnp.where(h < 4, 2 * h + 1, k - 1)
    e = jnp.minimum(idx, lim)
    e_ref[...] = e
    wv = jnp.sum(jnp.where(kiota == e, w, 0.0), axis=1, keepdims=True)
    q_ref[...] = x + (wv - x)
    ls = jnp.sum((wv - x) ** 2).reshape(1, 1)

    @pl.when(g == 0)
    def _():
        l_ref[...] = jnp.zeros_like(l_ref)

    l_ref[...] += ls


def kernel(input, weight):
    b, c, hgt, wdt = input.shape
    k = weight.shape[0]
    n = b * c * hgt * wdt
    x = input.reshape(n, 1)
    wt = weight.reshape(1, k)
    grid = n // _TN
    d, e, q, l = pl.pallas_call(
        functools.partial(_vq_block, hw=hgt * wdt, wdt=wdt, k=k),
        grid=(grid,),
        in_specs=[
            pl.BlockSpec((_TN, 1), lambda g: (g, 0)),
            pl.BlockSpec((1, k), lambda g: (0, 0)),
        ],
        out_specs=[
            pl.BlockSpec((_TN, k), lambda g: (g, 0)),
            pl.BlockSpec((_TN, 1), lambda g: (g, 0)),
            pl.BlockSpec((_TN, 1), lambda g: (g, 0)),
            pl.BlockSpec((1, 1), lambda g: (0, 0)),
        ],
        out_shape=[
            jax.ShapeDtypeStruct((n, k), jnp.float32),
            jax.ShapeDtypeStruct((n, 1), jnp.int32),
            jax.ShapeDtypeStruct((n, 1), jnp.float32),
            jax.ShapeDtypeStruct((1, 1), jnp.float32),
        ],
    )(x, wt)
    encoding = e.reshape(b, hgt, wdt)
    quantized_ste = q.reshape(b, c, hgt, wdt)
    loss = l[0, 0] * (2.0 / n)
    return quantized_ste, encoding, d, loss


# speedup vs baseline: 3.9298x; 1.1745x over previous
import functools

import jax
import jax.numpy as jnp
from jax import lax
from jax.experimental import pallas as pl

_TN = 1024


def _vq_block(x_ref, w_ref, fio_ref, d_ref, e_ref, q_ref, l_ref, *, hw, wdt, k):
    g = pl.program_id(0)
    x = x_ref[...]
    w = w_ref[...]
    fio = fio_ref[...]
    s = x * x + w * w
    d = s - x * (w + w)
    d_ref[...] = d
    mind = jnp.min(d, axis=1, keepdims=True)
    idxf = jnp.min(jnp.where(d == mind, fio, jnp.float32(k)), axis=1,
                   keepdims=True)
    idx = idxf.astype(jnp.int32)
    e_ref[...] = idx

    @pl.when((g % (hw // _TN)) == 0)
    def _():
        p = lax.broadcasted_iota(jnp.int32, (_TN, 1), 0)
        h = p // wdt
        lim = jnp.where(h < 4, 2 * h + 1, k - 1)
        e_ref[...] = jnp.minimum(idx, lim)

    e = e_ref[...]
    wv = jnp.sum(jnp.where(fio == e.astype(jnp.float32), w, 0.0), axis=1,
                 keepdims=True)
    q_ref[...] = x + (wv - x)
    ls = jnp.sum((wv - x) ** 2).reshape(1, 1)

    @pl.when(g == 0)
    def _():
        l_ref[...] = jnp.zeros_like(l_ref)

    l_ref[...] += ls


def kernel(input, weight):
    b, c, hgt, wdt = input.shape
    k = weight.shape[0]
    n = b * c * hgt * wdt
    x = input.reshape(n, 1)
    wt = weight.reshape(1, k)
    fio = jnp.arange(k, dtype=jnp.float32).reshape(1, k)
    grid = n // _TN
    d, e, q, l = pl.pallas_call(
        functools.partial(_vq_block, hw=hgt * wdt, wdt=wdt, k=k),
        grid=(grid,),
        in_specs=[
            pl.BlockSpec((_TN, 1), lambda g: (g, 0)),
            pl.BlockSpec((1, k), lambda g: (0, 0)),
            pl.BlockSpec((1, k), lambda g: (0, 0)),
        ],
        out_specs=[
            pl.BlockSpec((_TN, k), lambda g: (g, 0)),
            pl.BlockSpec((_TN, 1), lambda g: (g, 0)),
            pl.BlockSpec((_TN, 1), lambda g: (g, 0)),
            pl.BlockSpec((1, 1), lambda g: (0, 0)),
        ],
        out_shape=[
            jax.ShapeDtypeStruct((n, k), jnp.float32),
            jax.ShapeDtypeStruct((n, 1), jnp.int32),
            jax.ShapeDtypeStruct((n, 1), jnp.float32),
            jax.ShapeDtypeStruct((1, 1), jnp.float32),
        ],
    )(x, wt, fio)
    encoding = e.reshape(b, hgt, wdt)
    quantized_ste = q.reshape(b, c, hgt, wdt)
    loss = l[0, 0] * (2.0 / n)
    return quantized_ste, encoding, d, loss


# speedup vs baseline: 4.0188x; 1.0227x over previous
import functools

import jax
import jax.numpy as jnp
from jax import lax
from jax.experimental import pallas as pl
from jax.experimental.pallas import tpu as pltpu

_TN = 1024


def _vq_block(x_ref, w_ref, fio_ref, d_ref, e_ref, q_ref, l_ref, *, hw, wdt, k):
    g = pl.program_id(0)
    x = x_ref[...]
    w = w_ref[...]
    fio = fio_ref[...]
    s = x * x + w * w
    d = s - x * (w + w)
    d_ref[...] = d
    mind = jnp.min(d, axis=1, keepdims=True)
    mask = d == mind
    idxf = jnp.min(jnp.where(mask, fio, jnp.float32(k)), axis=1, keepdims=True)
    idx = idxf.astype(jnp.int32)
    e_ref[...] = idx
    wv = jnp.min(jnp.where(mask, w, jnp.float32(jnp.inf)), axis=1,
                 keepdims=True)
    q_ref[...] = x + (wv - x)

    @pl.when((g % (hw // _TN)) == 0)
    def _():
        p = lax.broadcasted_iota(jnp.int32, (_TN, 1), 0)
        h = p // wdt
        lim = jnp.where(h < 4, 2 * h + 1, k - 1)
        e = jnp.minimum(idx, lim)
        e_ref[...] = e
        wv2 = jnp.sum(jnp.where(fio == e.astype(jnp.float32), w, 0.0), axis=1,
                      keepdims=True)
        q_ref[...] = x + (wv2 - x)

    q = q_ref[...]
    l_ref[...] = jnp.sum((q - x) ** 2).reshape(1, 1, 1)


def kernel(input, weight):
    b, c, hgt, wdt = input.shape
    k = weight.shape[0]
    n = b * c * hgt * wdt
    x = input.reshape(n, 1)
    wt = weight.reshape(1, k)
    fio = jnp.arange(k, dtype=jnp.float32).reshape(1, k)
    grid = n // _TN
    d, e, q, l = pl.pallas_call(
        functools.partial(_vq_block, hw=hgt * wdt, wdt=wdt, k=k),
        grid=(grid,),
        in_specs=[
            pl.BlockSpec((_TN, 1), lambda g: (g, 0)),
            pl.BlockSpec((1, k), lambda g: (0, 0)),
            pl.BlockSpec((1, k), lambda g: (0, 0)),
        ],
        out_specs=[
            pl.BlockSpec((_TN, k), lambda g: (g, 0)),
            pl.BlockSpec((_TN, 1), lambda g: (g, 0)),
            pl.BlockSpec((_TN, 1), lambda g: (g, 0)),
            pl.BlockSpec((1, 1, 1), lambda g: (g, 0, 0)),
        ],
        out_shape=[
            jax.ShapeDtypeStruct((n, k), jnp.float32),
            jax.ShapeDtypeStruct((n, 1), jnp.int32),
            jax.ShapeDtypeStruct((n, 1), jnp.float32),
            jax.ShapeDtypeStruct((grid, 1, 1), jnp.float32),
        ],
        compiler_params=pltpu.CompilerParams(
            dimension_semantics=("parallel",)),
    )(x, wt, fio)
    encoding = e.reshape(b, hgt, wdt)
    quantized_ste = q.reshape(b, c, hgt, wdt)
    loss = jnp.sum(l) * (2.0 / n)
    return quantized_ste, encoding, d, loss


# speedup vs baseline: 4.7833x; 1.1902x over previous
import functools

import jax
import jax.numpy as jnp
from jax import lax
from jax.experimental import pallas as pl
from jax.experimental.pallas import tpu as pltpu

_TN = 3584


def _vq_block(x_ref, w_ref, fio_ref, d_ref, e_ref, q_ref, l_ref, *, hw, wdt, k):
    g = pl.program_id(0)
    x = x_ref[...]
    w = w_ref[...]
    fio = fio_ref[...]
    s = x * x + w * w
    d = s - x * (w + w)
    d_ref[...] = d
    mind = jnp.min(d, axis=1, keepdims=True)
    idxf = jnp.min(jnp.where(d == mind, fio, jnp.float32(k)), axis=1,
                   keepdims=True)
    idx = idxf.astype(jnp.int32)
    e_ref[...] = idx

    @pl.when((g % (hw // _TN)) == 0)
    def _():
        p = lax.broadcasted_iota(jnp.int32, (_TN, 1), 0)
        h = p // wdt
        lim = jnp.where(h < 4, 2 * h + 1, k - 1)
        e_ref[...] = jnp.minimum(idx, lim)

    ef = e_ref[...].astype(jnp.float32)
    wv = jnp.sum(jnp.where(fio == ef, w, 0.0), axis=1, keepdims=True)
    q_ref[...] = x + (wv - x)
    l_ref[...] = jnp.sum((wv - x) ** 2).reshape(1, 1, 1)


def kernel(input, weight):
    b, c, hgt, wdt = input.shape
    k = weight.shape[0]
    n = b * c * hgt * wdt
    x = input.reshape(n, 1)
    wt = weight.reshape(1, k)
    fio = jnp.arange(k, dtype=jnp.float32).reshape(1, k)
    grid = n // _TN
    d, e, q, l = pl.pallas_call(
        functools.partial(_vq_block, hw=hgt * wdt, wdt=wdt, k=k),
        grid=(grid,),
        in_specs=[
            pl.BlockSpec((_TN, 1), lambda g: (g, 0)),
            pl.BlockSpec((1, k), lambda g: (0, 0)),
            pl.BlockSpec((1, k), lambda g: (0, 0)),
        ],
        out_specs=[
            pl.BlockSpec((_TN, k), lambda g: (g, 0)),
            pl.BlockSpec((_TN, 1), lambda g: (g, 0)),
            pl.BlockSpec((_TN, 1), lambda g: (g, 0)),
            pl.BlockSpec((1, 1, 1), lambda g: (g, 0, 0)),
        ],
        out_shape=[
            jax.ShapeDtypeStruct((n, k), jnp.float32),
            jax.ShapeDtypeStruct((n, 1), jnp.int32),
            jax.ShapeDtypeStruct((n, 1), jnp.float32),
            jax.ShapeDtypeStruct((grid, 1, 1), jnp.float32),
        ],
        compiler_params=pltpu.CompilerParams(
            dimension_semantics=("parallel",)),
    )(x, wt, fio)
    encoding = e.reshape(b, hgt, wdt)
    quantized_ste = q.reshape(b, c, hgt, wdt)
    loss = jnp.sum(l) * (2.0 / n)
    return quantized_ste, encoding, d, loss
